# baseline (speedup 1.0000x reference)
import functools

import jax
import jax.numpy as jnp
from jax import lax
from jax.experimental import pallas as pl
from jax.experimental.pallas import tpu as pltpu
from jax.experimental.pallas import tpu_sc as plsc

N_CLS = 600
N_CTX = 5
D = 512
SEQ = 77
SUF = SEQ - 1 - N_CTX
B = 1024

NC = 2
NS = 16
NW = NC * NS
BPW = B // NW
LANES = 16
CHUNKS = D // LANES

ROW = SEQ * D
SUF_ROW = SUF * D
SUF_OFF = (1 + N_CTX) * D

_mesh = plsc.VectorSubcoreMesh(
    core_axis_name="c", subcore_axis_name="s", num_cores=NC, num_subcores=NS
)


@functools.partial(
    pl.kernel,
    out_type=jax.ShapeDtypeStruct((B, ROW), jnp.float32),
    mesh=_mesh,
    scratch_types=[
        pltpu.VMEM((BPW,), jnp.int32),
        pltpu.VMEM((BPW * 8,), jnp.int32),
        pltpu.VMEM((BPW, D), jnp.float32),
        pltpu.VMEM((BPW, D), jnp.float32),
        pltpu.VMEM((N_CTX, D), jnp.float32),
        pltpu.VMEM((1, ROW), jnp.float32),
        pltpu.SemaphoreType.DMA,
    ],
)
def _prompt_kernel(
    bias_hbm, target_hbm, target8_hbm, ctx_hbm, prefix_hbm, suffix_hbm, out_hbm,
    idx_v, idx8_v, bias_v, pre_v, ctx_v, stage_v, gsem,
):
    wid = lax.axis_index("s") * NC + lax.axis_index("c")
    base = wid * BPW

    pltpu.sync_copy(target_hbm.at[pl.ds(base, BPW)], idx_v)
    pltpu.sync_copy(target8_hbm.at[pl.ds(base * 8, BPW * 8)], idx8_v)
    pltpu.sync_copy(bias_hbm.at[pl.ds(base, BPW)], bias_v)
    pltpu.sync_copy(ctx_hbm, ctx_v)
    pltpu.async_copy(prefix_hbm.at[idx_v], pre_v, gsem).wait()

    def elem(i, carry):
        g = pltpu.async_copy(
            suffix_hbm.at[idx8_v.at[pl.ds(i * 8, 1)]],
            stage_v.at[:, pl.ds(SUF_OFF, SUF_ROW)],
            gsem,
        )

        def chunk(c, carry2):
            o = c * LANES
            stage_v[0, pl.ds(o, LANES)] = pre_v[i, pl.ds(o, LANES)]
            bb = bias_v[i, pl.ds(o, LANES)]
            for j in range(N_CTX):
                stage_v[0, pl.ds((1 + j) * D + o, LANES)] = (
                    ctx_v[j, pl.ds(o, LANES)] + bb
                )
            return carry2

        lax.fori_loop(0, CHUNKS, chunk, 0, unroll=2)

        g.wait()
        pltpu.sync_copy(stage_v, out_hbm.at[pl.ds(base + i, 1)])
        return carry

    lax.fori_loop(0, BPW, elem, 0)


def kernel(bias, target, ctx, token_prefix, token_suffix):
    target = target.astype(jnp.int32)
    target8 = jnp.repeat(target, 8)
    prefix2 = token_prefix.reshape(N_CLS, D)
    suffix2 = token_suffix.reshape(N_CLS, SUF_ROW)
    out = _prompt_kernel(bias, target, target8, ctx, prefix2, suffix2)
    return out.reshape(B, SEQ, D)


# speedup vs baseline: 1.0420x; 1.0420x over previous
import functools

import jax
import jax.numpy as jnp
from jax import lax
from jax.experimental import pallas as pl
from jax.experimental.pallas import tpu as pltpu
from jax.experimental.pallas import tpu_sc as plsc

N_CLS = 600
N_CTX = 5
D = 512
SEQ = 77
SUF = SEQ - 1 - N_CTX
B = 1024

NC = 2
NS = 16
NW = NC * NS
BPW = B // NW
LANES = 16
CHUNKS = D // LANES

ROW = SEQ * D
SUF_ROW = SUF * D
SUF_OFF = (1 + N_CTX) * D

_mesh = plsc.VectorSubcoreMesh(
    core_axis_name="c", subcore_axis_name="s", num_cores=NC, num_subcores=NS
)


@functools.partial(
    pl.kernel,
    out_type=jax.ShapeDtypeStruct((B, ROW), jnp.float32),
    mesh=_mesh,
    scratch_types=[
        pltpu.VMEM((BPW,), jnp.int32),
        pltpu.VMEM((BPW * 8,), jnp.int32),
        pltpu.VMEM((BPW, D), jnp.float32),
        pltpu.VMEM((BPW, D), jnp.float32),
        pltpu.VMEM((N_CTX, D), jnp.float32),
        pltpu.VMEM((1, ROW), jnp.float32),
        pltpu.VMEM((1, ROW), jnp.float32),
        pltpu.SemaphoreType.DMA,
        pltpu.SemaphoreType.DMA,
    ],
)
def _prompt_kernel(
    bias_hbm, target_hbm, target8_hbm, ctx_hbm, prefix_hbm, suffix_hbm, out_hbm,
    idx_v, idx8_v, bias_v, pre_v, ctx_v, stage0_v, stage1_v, gsem, osem,
):
    stages = (stage0_v, stage1_v)
    wid = lax.axis_index("s") * NC + lax.axis_index("c")
    base = wid * BPW

    pltpu.sync_copy(target_hbm.at[pl.ds(base, BPW)], idx_v)
    pltpu.sync_copy(target8_hbm.at[pl.ds(base * 8, BPW * 8)], idx8_v)
    pltpu.sync_copy(bias_hbm.at[pl.ds(base, BPW)], bias_v)
    pltpu.sync_copy(ctx_hbm, ctx_v)
    pltpu.async_copy(prefix_hbm.at[idx_v], pre_v, gsem).wait()

    def fire_gather(i, bf):
        return pltpu.async_copy(
            suffix_hbm.at[idx8_v.at[pl.ds(i * 8, 1)]],
            stages[bf].at[:, pl.ds(SUF_OFF, SUF_ROW)],
            gsem,
        )

    def compute_head(i, bf):
        def chunk(c, carry2):
            o = c * LANES
            stages[bf][0, pl.ds(o, LANES)] = pre_v[i, pl.ds(o, LANES)]
            bb = bias_v[i, pl.ds(o, LANES)]
            for j in range(N_CTX):
                stages[bf][0, pl.ds((1 + j) * D + o, LANES)] = (
                    ctx_v[j, pl.ds(o, LANES)] + bb
                )
            return carry2

        lax.fori_loop(0, CHUNKS, chunk, 0, unroll=2)

    def fire_out(i, bf):
        pltpu.async_copy(stages[bf], out_hbm.at[pl.ds(base + i, 1)], osem)

    def wait_out_one(bf):
        pltpu.make_async_copy(
            stages[bf], out_hbm.at[pl.ds(base, 1)], osem
        ).wait()

    for bf in (0, 1):
        g = fire_gather(bf, bf)
        compute_head(bf, bf)
        g.wait()
        fire_out(bf, bf)

    def pair(k, carry):
        g0 = k * 2
        for bf in (0, 1):
            i = g0 + bf
            wait_out_one(bf)
            g = fire_gather(i, bf)
            compute_head(i, bf)
            g.wait()
            fire_out(i, bf)
        return carry

    lax.fori_loop(1, BPW // 2, pair, 0)

    wait_out_one(0)
    wait_out_one(1)


def kernel(bias, target, ctx, token_prefix, token_suffix):
    target = target.astype(jnp.int32)
    target8 = jnp.repeat(target, 8)
    prefix2 = token_prefix.reshape(N_CLS, D)
    suffix2 = token_suffix.reshape(N_CLS, SUF_ROW)
    out = _prompt_kernel(bias, target, target8, ctx, prefix2, suffix2)
    return out.reshape(B, SEQ, D)
